# baseline (speedup 1.0000x reference)
import functools

import jax
import jax.numpy as jnp
import numpy as np
from jax import lax
from jax.experimental import pallas as pl
from jax.experimental.pallas import tpu as pltpu
from jax.experimental.pallas import tpu_sc as plsc

ROWS = 16384
IN_COLS = 2048
OUT_COLS = 1024

NUM_WORKERS = 32
ROWS_PER_WORKER = ROWS // NUM_WORKERS
CHUNK_ROWS = 8
NBUF = 4
CHUNKS_PER_WORKER = ROWS_PER_WORKER // CHUNK_ROWS
LANES = 16

_CHOICE = np.array([1308, 98, 1494, 1367, 1392, 726, 410, 1311, 1631, 1841, 360, 1261, 1990, 139, 467, 1964, 1122, 1547, 739, 892, 198, 610, 1721, 1669, 1822, 1265, 1502, 1965, 858, 292, 210, 965, 1029, 1185, 1888, 1968, 688, 1230, 941, 158, 352, 539, 294, 795, 26, 919, 120, 853, 216, 340, 1356, 1324, 1164, 236, 13, 482, 414, 1168, 1726, 1854, 873, 883, 1909, 1982, 73, 90, 107, 953, 114, 752, 1388, 1274, 1556, 702, 88, 226, 868, 1707, 49, 488, 1761, 1248, 423, 442, 641, 1767, 1755, 1012, 1570, 1598, 0, 1111, 855, 1142, 1713, 601, 529, 34, 1522, 1187, 305, 1087, 202, 948, 751, 443, 806, 206, 1067, 803, 637, 250, 1224, 51, 1147, 1772, 533, 457, 661, 1402, 863, 242, 1534, 1366, 666, 1756, 1445, 622, 709, 437, 519, 142, 1847, 1658, 95, 1700, 1863, 1381, 1042, 991, 75, 357, 794, 1549, 495, 1614, 1451, 525, 1262, 1030, 1925, 1904, 404, 1680, 1942, 200, 385, 1134, 239, 2003, 39, 619, 1327, 459, 680, 1475, 432, 694, 1518, 141, 588, 685, 1660, 122, 715, 1783, 35, 1139, 274, 797, 1346, 608, 670, 2001, 362, 409, 1428, 978, 658, 1543, 1341, 1343, 1708, 958, 1843, 1440, 1406, 378, 1719, 341, 123, 1306, 116, 1107, 1967, 21, 1781, 1896, 1056, 1026, 551, 1450, 1926, 1711, 370, 649, 268, 307, 2034, 2011, 168, 1500, 1739, 2000, 1218, 503, 1325, 748, 1616, 1193, 1605, 1437, 1319, 1595, 1427, 252, 1481, 1851, 1116, 1102, 902, 4, 1053, 273, 1098, 600, 1453, 386, 1927, 1734, 1859, 1974, 1221, 1683, 763, 1532, 1724, 365, 829, 732, 1277, 1831, 1439, 586, 890, 1836, 96, 1656, 581, 230, 900, 1943, 1498, 416, 1, 1794, 1106, 152, 520, 827, 969, 1206, 245, 1624, 1741, 452, 1803, 129, 549, 76, 924, 857, 1931, 884, 623, 1174, 558, 862, 1826, 315, 448, 361, 754, 1559, 568, 1586, 254, 1035, 952, 81, 769, 41, 1144, 2018, 501, 248, 1268, 382, 575, 1899, 1104, 2019, 1213, 1489, 338, 1045, 973, 280, 1121, 255, 1099, 1579, 954, 1555, 1061, 921, 89, 1090, 1569, 422, 1635, 400, 93, 1241, 1373, 407, 1079, 205, 209, 363, 1988, 839, 636, 871, 647, 1796, 698, 1048, 615, 218, 1186, 894, 434, 1393, 767, 1088, 672, 1084, 47, 692, 293, 66, 1845, 70, 756, 174, 222, 1457, 2014, 532, 1520, 1821, 1645, 1077, 1488, 1149, 793, 1097, 1001, 1671, 1618, 1505, 1811, 1156, 387, 1685, 1674, 426, 1008, 128, 617, 882, 980, 648, 1524, 1996, 1938, 1597, 194, 1834, 1467, 1949, 1289, 1743, 312, 1833, 1615, 1305, 1027, 1095, 1177, 598, 1212, 393, 1897, 1986, 1485, 917, 285, 1940, 321, 347, 1566, 950, 1966, 1062, 611, 728, 1257, 11, 1426, 1307, 1676, 435, 1873, 984, 1696, 1083, 1215, 741, 1960, 625, 1419, 845, 1345, 1535, 308, 309, 1171, 572, 779, 785, 1571, 824, 557, 1916, 1359, 578, 156, 771, 440, 1058, 430, 706, 1805, 9, 1123, 1023, 1145, 244, 1663, 1161, 1878, 1934, 1880, 1483, 1544, 997, 1234, 681, 1094, 727, 1179, 1376, 1953, 492, 1499, 995, 718, 736, 333, 1792, 1390, 1000, 1868, 1253, 1205, 957, 1014, 345, 787, 961, 1906, 1944, 498, 59, 1918, 303, 1291, 994, 789, 306, 1060, 1496, 2025, 1347, 1735, 712, 913, 1162, 322, 605, 1175, 696, 1638, 1923, 2035, 469, 1619, 1632, 1052, 2036, 104, 155, 1143, 1070, 679, 554, 1389, 990, 1908, 1991, 955, 1917, 477, 241, 840, 1507, 1120, 583, 584, 2020, 286, 282, 324, 375, 1091, 1812, 401, 471, 50, 776, 1935, 311, 881, 872, 464, 1609, 1462, 1071, 36, 550, 1242, 1076, 1993, 58, 521, 1890, 1446, 1290, 559, 505, 203, 344, 251, 669, 1247, 468, 447, 676, 329, 395, 1133, 1703, 381, 2045, 176, 1266, 846, 1448, 906, 1759, 677, 1819, 1576, 1046, 1217, 1649, 454, 461, 589, 527, 1806, 153, 481, 1384, 462, 368, 118, 1281, 173, 962, 1521, 1220, 1970, 97, 655, 1928, 74, 472, 1865, 590, 358, 916, 1973, 1320, 1853, 1279, 33, 67, 438, 1705, 877, 909, 656, 121, 1041, 542, 690, 94, 424, 1434, 1804, 885, 290, 1447, 1848, 770, 800, 1286, 335, 592, 2041, 1983, 1314, 1210, 53, 1687, 1348, 64, 1684, 1225, 939, 1932, 889, 1201, 1747, 1478, 1517, 1647, 154, 1946, 148, 1044, 711, 259, 1736, 453, 342, 1501, 211, 936, 1016, 534, 327, 528, 725, 878, 1477, 1670, 1117, 697, 1137, 582, 110, 1894, 1950, 1280, 316, 657, 1140, 337, 1563, 1387, 1442, 970, 1539, 1737, 1930, 1011, 48, 1131, 993, 1157, 911, 1976, 379, 644, 112, 1113, 1602, 1471, 1444, 1951, 573, 1929, 460, 838, 1907, 1295, 246, 40, 1695, 1115, 742, 875, 111, 412, 1648, 1745, 159, 1655, 1013, 1249, 1738, 814, 1617, 1560, 1129, 691, 546, 1902, 1441, 301, 466, 1303, 804, 1731, 243, 502, 1552, 1704, 844, 1654, 1433, 1766, 999, 170, 1251, 1958, 1391, 102, 515, 1611, 843, 531, 1302, 2028, 587, 483, 1125, 1336, 392, 297, 348, 1893, 1898, 220, 1877, 1508, 2039, 979, 31, 1321, 221, 319, 1329, 476, 1613, 576, 470, 1798, 1255, 1709, 1369, 1370, 1955, 1797, 1725, 635, 69, 1577, 1591, 1410, 1978, 295, 1059, 389, 825, 836, 960, 402, 928, 237, 1723, 484, 1795, 1342, 989, 1228, 1411, 22, 191, 1528, 223, 1153, 1417, 992, 1470, 1354, 543, 629, 998, 1554, 182, 1722, 195, 1533, 1855, 1551, 1317, 848, 1901, 905, 172, 1921, 1860, 330, 1947, 313, 1782, 227, 719, 895, 1565, 1604, 1039, 856, 1422, 1178, 238, 634, 737, 1191, 1583, 108, 351, 1531, 1748, 602, 790, 2047, 523, 78, 1004, 1380, 2042, 1033, 213, 1939, 1232, 645, 479, 275, 166, 1260, 1913, 418, 383, 982, 1297, 25, 1775, 512, 1646, 959, 609, 1607, 1304, 834, 1937, 1009, 792, 553, 371, 652, 1665, 16, 1810, 567, 217, 1032, 1207, 1692, 933, 72, 1425, 796, 612, 1181, 1360, 570, 1100, 721, 374, 1874, 284, 1962, 977, 1786, 486, 1066, 1801, 577, 510, 816, 1337, 177, 1657, 1283, 2008, 1673, 463, 1264, 1017, 556, 1105, 119, 1007, 1542, 1486, 1817, 289, 1491, 3, 1195, 693, 1294, 146, 701, 654, 975, 1429, 2007, 1003, 942, 915, 1288, 1779, 837, 229, 607, 126, 1636, 540, 1884, 184, 922, 1364, 1093, 1809, 1051, 1562, 621, 594, 113, 1513, 1235, 805, 1784, 23, 1377, 272, 760], dtype=np.int32)


def _sc_body(x_hbm, idx_hbm, out_hbm, idx_v,
             in_bufs, out_bufs, in_sems, out_sems):
    c = lax.axis_index("c")
    s = lax.axis_index("s")
    wid = s * 2 + c
    base = wid * ROWS_PER_WORKER

    def in_copy(ci, b):
        row0 = base + ci * CHUNK_ROWS
        return pltpu.make_async_copy(
            x_hbm.at[pl.ds(row0, CHUNK_ROWS)], in_bufs[b], in_sems[b])

    def out_copy(ci, b):
        row0 = base + ci * CHUNK_ROWS
        return pltpu.make_async_copy(
            out_bufs[b], out_hbm.at[pl.ds(row0, CHUNK_ROWS)], out_sems[b])

    def gather_chunk(b):
        in_buf = in_bufs[b]
        out_buf = out_bufs[b]

        @plsc.parallel_loop(0, OUT_COLS, step=LANES, unroll=4)
        def j_body(j0):
            col = idx_v[pl.ds(j0, LANES)]
            for r in range(CHUNK_ROWS):
                row_idx = jnp.full((LANES,), r, jnp.int32)
                v = plsc.load_gather(in_buf, [row_idx, col])
                out_buf[r, pl.ds(j0, LANES)] = v

    for b in range(NBUF):
        in_copy(b, b).start()
    pltpu.sync_copy(idx_hbm, idx_v)

    for b in range(NBUF):
        in_copy(b, b).wait()
        gather_chunk(b)
        out_copy(b, b).start()
        in_copy(NBUF + b, b).start()

    def loop_body(ci, carry):
        for b in range(NBUF):
            in_copy(ci + b, b).wait()
            out_copy(ci + b, b).wait()
            gather_chunk(b)
            out_copy(ci + b, b).start()
            in_copy(ci + NBUF + b, b).start()
        return carry

    lax.fori_loop(1, CHUNKS_PER_WORKER // NBUF - 1,
                  lambda i, cr: loop_body(i * NBUF, cr), 0)

    last = CHUNKS_PER_WORKER - NBUF
    for b in range(NBUF):
        in_copy(last + b, b).wait()
        out_copy(last + b, b).wait()
        gather_chunk(b)
        out_copy(last + b, b).start()
    for b in range(NBUF):
        out_copy(0, b).wait()


@functools.partial(jax.jit, static_argnums=())
def kernel(x):
    mesh = plsc.VectorSubcoreMesh(core_axis_name="c", subcore_axis_name="s")
    run = pl.kernel(
        _sc_body,
        out_type=jax.ShapeDtypeStruct((ROWS, OUT_COLS), jnp.float32),
        mesh=mesh,
        scratch_types=[
            pltpu.VMEM((OUT_COLS,), jnp.int32),
            tuple(pltpu.VMEM((CHUNK_ROWS, IN_COLS), jnp.float32)
                  for _ in range(NBUF)),
            tuple(pltpu.VMEM((CHUNK_ROWS, OUT_COLS), jnp.float32)
                  for _ in range(NBUF)),
            tuple(pltpu.SemaphoreType.DMA for _ in range(NBUF)),
            tuple(pltpu.SemaphoreType.DMA for _ in range(NBUF)),
        ],
        compiler_params=pltpu.CompilerParams(needs_layout_passes=False),
    )
    return run(x, jnp.asarray(_CHOICE))


# speedup vs baseline: 1.0283x; 1.0283x over previous
import functools

import jax
import jax.numpy as jnp
import numpy as np
from jax import lax
from jax.experimental import pallas as pl
from jax.experimental.pallas import tpu as pltpu
from jax.experimental.pallas import tpu_sc as plsc

ROWS = 16384
IN_COLS = 2048
OUT_COLS = 1024

NUM_WORKERS = 32
ROWS_PER_WORKER = ROWS // NUM_WORKERS
CHUNK_ROWS = 8
NBUF = 4
CHUNKS_PER_WORKER = ROWS_PER_WORKER // CHUNK_ROWS
LANES = 16

_CHOICE = np.array([1308, 98, 1494, 1367, 1392, 726, 410, 1311, 1631, 1841, 360, 1261, 1990, 139, 467, 1964, 1122, 1547, 739, 892, 198, 610, 1721, 1669, 1822, 1265, 1502, 1965, 858, 292, 210, 965, 1029, 1185, 1888, 1968, 688, 1230, 941, 158, 352, 539, 294, 795, 26, 919, 120, 853, 216, 340, 1356, 1324, 1164, 236, 13, 482, 414, 1168, 1726, 1854, 873, 883, 1909, 1982, 73, 90, 107, 953, 114, 752, 1388, 1274, 1556, 702, 88, 226, 868, 1707, 49, 488, 1761, 1248, 423, 442, 641, 1767, 1755, 1012, 1570, 1598, 0, 1111, 855, 1142, 1713, 601, 529, 34, 1522, 1187, 305, 1087, 202, 948, 751, 443, 806, 206, 1067, 803, 637, 250, 1224, 51, 1147, 1772, 533, 457, 661, 1402, 863, 242, 1534, 1366, 666, 1756, 1445, 622, 709, 437, 519, 142, 1847, 1658, 95, 1700, 1863, 1381, 1042, 991, 75, 357, 794, 1549, 495, 1614, 1451, 525, 1262, 1030, 1925, 1904, 404, 1680, 1942, 200, 385, 1134, 239, 2003, 39, 619, 1327, 459, 680, 1475, 432, 694, 1518, 141, 588, 685, 1660, 122, 715, 1783, 35, 1139, 274, 797, 1346, 608, 670, 2001, 362, 409, 1428, 978, 658, 1543, 1341, 1343, 1708, 958, 1843, 1440, 1406, 378, 1719, 341, 123, 1306, 116, 1107, 1967, 21, 1781, 1896, 1056, 1026, 551, 1450, 1926, 1711, 370, 649, 268, 307, 2034, 2011, 168, 1500, 1739, 2000, 1218, 503, 1325, 748, 1616, 1193, 1605, 1437, 1319, 1595, 1427, 252, 1481, 1851, 1116, 1102, 902, 4, 1053, 273, 1098, 600, 1453, 386, 1927, 1734, 1859, 1974, 1221, 1683, 763, 1532, 1724, 365, 829, 732, 1277, 1831, 1439, 586, 890, 1836, 96, 1656, 581, 230, 900, 1943, 1498, 416, 1, 1794, 1106, 152, 520, 827, 969, 1206, 245, 1624, 1741, 452, 1803, 129, 549, 76, 924, 857, 1931, 884, 623, 1174, 558, 862, 1826, 315, 448, 361, 754, 1559, 568, 1586, 254, 1035, 952, 81, 769, 41, 1144, 2018, 501, 248, 1268, 382, 575, 1899, 1104, 2019, 1213, 1489, 338, 1045, 973, 280, 1121, 255, 1099, 1579, 954, 1555, 1061, 921, 89, 1090, 1569, 422, 1635, 400, 93, 1241, 1373, 407, 1079, 205, 209, 363, 1988, 839, 636, 871, 647, 1796, 698, 1048, 615, 218, 1186, 894, 434, 1393, 767, 1088, 672, 1084, 47, 692, 293, 66, 1845, 70, 756, 174, 222, 1457, 2014, 532, 1520, 1821, 1645, 1077, 1488, 1149, 793, 1097, 1001, 1671, 1618, 1505, 1811, 1156, 387, 1685, 1674, 426, 1008, 128, 617, 882, 980, 648, 1524, 1996, 1938, 1597, 194, 1834, 1467, 1949, 1289, 1743, 312, 1833, 1615, 1305, 1027, 1095, 1177, 598, 1212, 393, 1897, 1986, 1485, 917, 285, 1940, 321, 347, 1566, 950, 1966, 1062, 611, 728, 1257, 11, 1426, 1307, 1676, 435, 1873, 984, 1696, 1083, 1215, 741, 1960, 625, 1419, 845, 1345, 1535, 308, 309, 1171, 572, 779, 785, 1571, 824, 557, 1916, 1359, 578, 156, 771, 440, 1058, 430, 706, 1805, 9, 1123, 1023, 1145, 244, 1663, 1161, 1878, 1934, 1880, 1483, 1544, 997, 1234, 681, 1094, 727, 1179, 1376, 1953, 492, 1499, 995, 718, 736, 333, 1792, 1390, 1000, 1868, 1253, 1205, 957, 1014, 345, 787, 961, 1906, 1944, 498, 59, 1918, 303, 1291, 994, 789, 306, 1060, 1496, 2025, 1347, 1735, 712, 913, 1162, 322, 605, 1175, 696, 1638, 1923, 2035, 469, 1619, 1632, 1052, 2036, 104, 155, 1143, 1070, 679, 554, 1389, 990, 1908, 1991, 955, 1917, 477, 241, 840, 1507, 1120, 583, 584, 2020, 286, 282, 324, 375, 1091, 1812, 401, 471, 50, 776, 1935, 311, 881, 872, 464, 1609, 1462, 1071, 36, 550, 1242, 1076, 1993, 58, 521, 1890, 1446, 1290, 559, 505, 203, 344, 251, 669, 1247, 468, 447, 676, 329, 395, 1133, 1703, 381, 2045, 176, 1266, 846, 1448, 906, 1759, 677, 1819, 1576, 1046, 1217, 1649, 454, 461, 589, 527, 1806, 153, 481, 1384, 462, 368, 118, 1281, 173, 962, 1521, 1220, 1970, 97, 655, 1928, 74, 472, 1865, 590, 358, 916, 1973, 1320, 1853, 1279, 33, 67, 438, 1705, 877, 909, 656, 121, 1041, 542, 690, 94, 424, 1434, 1804, 885, 290, 1447, 1848, 770, 800, 1286, 335, 592, 2041, 1983, 1314, 1210, 53, 1687, 1348, 64, 1684, 1225, 939, 1932, 889, 1201, 1747, 1478, 1517, 1647, 154, 1946, 148, 1044, 711, 259, 1736, 453, 342, 1501, 211, 936, 1016, 534, 327, 528, 725, 878, 1477, 1670, 1117, 697, 1137, 582, 110, 1894, 1950, 1280, 316, 657, 1140, 337, 1563, 1387, 1442, 970, 1539, 1737, 1930, 1011, 48, 1131, 993, 1157, 911, 1976, 379, 644, 112, 1113, 1602, 1471, 1444, 1951, 573, 1929, 460, 838, 1907, 1295, 246, 40, 1695, 1115, 742, 875, 111, 412, 1648, 1745, 159, 1655, 1013, 1249, 1738, 814, 1617, 1560, 1129, 691, 546, 1902, 1441, 301, 466, 1303, 804, 1731, 243, 502, 1552, 1704, 844, 1654, 1433, 1766, 999, 170, 1251, 1958, 1391, 102, 515, 1611, 843, 531, 1302, 2028, 587, 483, 1125, 1336, 392, 297, 348, 1893, 1898, 220, 1877, 1508, 2039, 979, 31, 1321, 221, 319, 1329, 476, 1613, 576, 470, 1798, 1255, 1709, 1369, 1370, 1955, 1797, 1725, 635, 69, 1577, 1591, 1410, 1978, 295, 1059, 389, 825, 836, 960, 402, 928, 237, 1723, 484, 1795, 1342, 989, 1228, 1411, 22, 191, 1528, 223, 1153, 1417, 992, 1470, 1354, 543, 629, 998, 1554, 182, 1722, 195, 1533, 1855, 1551, 1317, 848, 1901, 905, 172, 1921, 1860, 330, 1947, 313, 1782, 227, 719, 895, 1565, 1604, 1039, 856, 1422, 1178, 238, 634, 737, 1191, 1583, 108, 351, 1531, 1748, 602, 790, 2047, 523, 78, 1004, 1380, 2042, 1033, 213, 1939, 1232, 645, 479, 275, 166, 1260, 1913, 418, 383, 982, 1297, 25, 1775, 512, 1646, 959, 609, 1607, 1304, 834, 1937, 1009, 792, 553, 371, 652, 1665, 16, 1810, 567, 217, 1032, 1207, 1692, 933, 72, 1425, 796, 612, 1181, 1360, 570, 1100, 721, 374, 1874, 284, 1962, 977, 1786, 486, 1066, 1801, 577, 510, 816, 1337, 177, 1657, 1283, 2008, 1673, 463, 1264, 1017, 556, 1105, 119, 1007, 1542, 1486, 1817, 289, 1491, 3, 1195, 693, 1294, 146, 701, 654, 975, 1429, 2007, 1003, 942, 915, 1288, 1779, 837, 229, 607, 126, 1636, 540, 1884, 184, 922, 1364, 1093, 1809, 1051, 1562, 621, 594, 113, 1513, 1235, 805, 1784, 23, 1377, 272, 760], dtype=np.int32)


def _sc_body(x_hbm, idx_hbm, out_hbm, idx_v,
             in_bufs, out_bufs, in_sems, out_sems):
    c = lax.axis_index("c")
    s = lax.axis_index("s")
    wid = s * 2 + c
    base = wid * ROWS_PER_WORKER

    def in_copy(ci, b):
        row0 = base + ci * CHUNK_ROWS
        return pltpu.make_async_copy(
            x_hbm.at[pl.ds(row0, CHUNK_ROWS)], in_bufs[b], in_sems[b])

    def out_copy(ci, b):
        row0 = base + ci * CHUNK_ROWS
        return pltpu.make_async_copy(
            out_bufs[b], out_hbm.at[pl.ds(row0, CHUNK_ROWS)], out_sems[b])

    def gather_chunk(b):
        in_buf = in_bufs[b]
        out_buf = out_bufs[b]

        @plsc.parallel_loop(0, OUT_COLS, step=LANES, unroll=4)
        def j_body(j0):
            col = idx_v[pl.ds(j0, LANES)]
            for r in range(CHUNK_ROWS):
                row_idx = jnp.full((LANES,), r, jnp.int32)
                v = plsc.load_gather(in_buf, [row_idx, col])
                out_buf[r, pl.ds(j0, LANES)] = v

    for b in range(NBUF):
        in_copy(b, b).start()
    pltpu.sync_copy(idx_hbm, idx_v)

    def loop_body(ci, carry):
        for b in range(NBUF):
            in_copy(ci + b, b).wait()

            @pl.when(ci > 0)
            def _():
                out_copy(ci + b, b).wait()

            gather_chunk(b)
            out_copy(ci + b, b).start()

            @pl.when(ci + NBUF + b < CHUNKS_PER_WORKER)
            def _():
                in_copy(ci + NBUF + b, b).start()
        return carry

    lax.fori_loop(0, CHUNKS_PER_WORKER // NBUF,
                  lambda i, cr: loop_body(i * NBUF, cr), 0)
    for b in range(NBUF):
        out_copy(0, b).wait()


@functools.partial(jax.jit, static_argnums=())
def kernel(x):
    mesh = plsc.VectorSubcoreMesh(core_axis_name="c", subcore_axis_name="s")
    run = pl.kernel(
        _sc_body,
        out_type=jax.ShapeDtypeStruct((ROWS, OUT_COLS), jnp.float32),
        mesh=mesh,
        scratch_types=[
            pltpu.VMEM((OUT_COLS,), jnp.int32),
            tuple(pltpu.VMEM((CHUNK_ROWS, IN_COLS), jnp.float32)
                  for _ in range(NBUF)),
            tuple(pltpu.VMEM((CHUNK_ROWS, OUT_COLS), jnp.float32)
                  for _ in range(NBUF)),
            tuple(pltpu.SemaphoreType.DMA for _ in range(NBUF)),
            tuple(pltpu.SemaphoreType.DMA for _ in range(NBUF)),
        ],
        compiler_params=pltpu.CompilerParams(needs_layout_passes=False),
    )
    return run(x, jnp.asarray(_CHOICE))
